# baseline (speedup 1.0000x reference)
import functools
import jax
import jax.numpy as jnp
from jax import lax
from jax.experimental import pallas as pl
from jax.experimental.pallas import tpu as pltpu
from jax.experimental.pallas import tpu_sc as plsc

_NC = 2
_NS = 16
_NW = _NC * _NS
_CHUNK = 80
_ZROWS = 128
_NPAD = 10240


_DCHUNK = 400


def _deg_body(n, epw, src_hbm, dst_hbm, z_hbm, outs_hbm, outd_hbm,
              ia0, ia1, ib0, ib1, sa0, sa1, sb0, sb1,
              hist_s, hist_d):
    c = lax.axis_index("c")
    s = lax.axis_index("s")
    wid = s * _NC + c

    pltpu.sync_copy(z_hbm, hist_s)
    pltpu.sync_copy(z_hbm, hist_d)

    base = wid * epw
    nchunks = epw // _DCHUNK
    ia = (ia0, ia1)
    ib = (ib0, ib1)
    sa = (sa0, sa1)
    sb = (sb0, sb1)
    ones = jnp.full((16,), 1.0, jnp.float32)

    def load(p, ci):
        off = pl.multiple_of(base + ci * _DCHUNK, 8)
        pltpu.async_copy(src_hbm.at[pl.ds(off, _DCHUNK)], ia[p], sa[p])
        pltpu.async_copy(dst_hbm.at[pl.ds(off, _DCHUNK)], ib[p], sb[p])

    def wait_load(p):
        pltpu.make_async_copy(src_hbm.at[pl.ds(0, _DCHUNK)], ia[p], sa[p]).wait()
        pltpu.make_async_copy(dst_hbm.at[pl.ds(0, _DCHUNK)], ib[p], sb[p]).wait()

    def process(p):
        for j in range(_DCHUNK // 16):
            plsc.addupdate_scatter(hist_s, [ia[p][pl.ds(j * 16, 16)]], ones)
            plsc.addupdate_scatter(hist_d, [ib[p][pl.ds(j * 16, 16)]], ones)

    load(0, 0)

    @pl.loop(0, (nchunks - 1) // 2)
    def _(gi):
        for k in range(2):
            ci = gi * 2 + k
            wait_load(k)
            load(1 - k, ci + 1)
            process(k)

    wait_load(0)
    process(0)

    pltpu.sync_copy(hist_s, outs_hbm.at[pl.ds(wid * n, n)])
    pltpu.sync_copy(hist_d, outd_hbm.at[pl.ds(wid * n, n)])


def _degrees(src, dst, n):
    e = src.shape[0]
    epw = e // _NW
    assert epw % _DCHUNK == 0 and (epw // _DCHUNK) % 2 == 1
    z = jnp.zeros((n,), jnp.float32)
    mesh = plsc.VectorSubcoreMesh(core_axis_name="c", subcore_axis_name="s")
    f = pl.kernel(
        functools.partial(_deg_body, n, epw),
        out_type=[
            jax.ShapeDtypeStruct((_NW * n,), jnp.float32),
            jax.ShapeDtypeStruct((_NW * n,), jnp.float32),
        ],
        mesh=mesh,
        scratch_types=[
            pltpu.VMEM((_DCHUNK,), jnp.int32),
            pltpu.VMEM((_DCHUNK,), jnp.int32),
            pltpu.VMEM((_DCHUNK,), jnp.int32),
            pltpu.VMEM((_DCHUNK,), jnp.int32),
            pltpu.SemaphoreType.DMA,
            pltpu.SemaphoreType.DMA,
            pltpu.SemaphoreType.DMA,
            pltpu.SemaphoreType.DMA,
            pltpu.VMEM((n,), jnp.float32),
            pltpu.VMEM((n,), jnp.float32),
        ],
        compiler_params=pltpu.CompilerParams(needs_layout_passes=False),
    )
    outs, outd = f(src, dst, z)
    return outs.reshape(_NW, n), outd.reshape(_NW, n)


_ACHUNK = 128
_ATAIL = 16


def _agg_body(epw, x_hbm, src_hbm, dst_hbm, z_hbm, out_hbm,
              idx_s0, idx_s1, idx_s2, idx_s3, idx_d0, idx_d1, idx_d2, idx_d3,
              rows0, rows1, idx_ts, idx_td, rows_t,
              sem_g0, sem_g1, sem_s0, sem_s1,
              sem_i0, sem_i1, sem_i2, sem_i3, sem_t, acc):
    c = lax.axis_index("c")
    s = lax.axis_index("s")
    wid = s * _NC + c
    nps = _NPAD // _NS

    pltpu.sync_copy(z_hbm, acc.at[pl.ds(s * nps, nps)])
    plsc.subcore_barrier()

    base = wid * epw
    nchunks = epw // _ACHUNK
    idx_s = (idx_s0, idx_s1, idx_s2, idx_s3)
    idx_d = (idx_d0, idx_d1, idx_d2, idx_d3)
    rows = (rows0, rows1)
    sem_g = (sem_g0, sem_g1)
    sem_s = (sem_s0, sem_s1)
    sem_i = (sem_i0, sem_i1, sem_i2, sem_i3)

    def load_idx(q, ci):
        off = pl.multiple_of(base + ci * _ACHUNK, 8)
        pltpu.async_copy(src_hbm.at[pl.ds(off, _ACHUNK)], idx_s[q], sem_i[q])
        pltpu.async_copy(dst_hbm.at[pl.ds(off, _ACHUNK)], idx_d[q], sem_i[q])

    def wait_idx(q):
        pltpu.make_async_copy(src_hbm.at[pl.ds(0, _ACHUNK)], idx_s[q],
                              sem_i[q]).wait()
        pltpu.make_async_copy(dst_hbm.at[pl.ds(0, _ACHUNK)], idx_d[q],
                              sem_i[q]).wait()

    def gather(p, q):
        pltpu.async_copy(x_hbm.at[idx_s[q]], rows[p], sem_g[p])

    def wait_gather(p, q):
        pltpu.make_async_copy(x_hbm.at[idx_s[q]], rows[p], sem_g[p]).wait()

    def scatter(p, q):
        pltpu.async_copy(rows[p], acc.at[idx_d[q]], sem_s[p], add=True)

    def wait_scatter(p, q):
        pltpu.make_async_copy(rows[p], acc.at[idx_d[q]], sem_s[p]).wait()

    toff = pl.multiple_of(base + nchunks * _ACHUNK, 8)
    pltpu.sync_copy(src_hbm.at[pl.ds(toff, _ATAIL)], idx_ts)
    pltpu.sync_copy(dst_hbm.at[pl.ds(toff, _ATAIL)], idx_td)
    pltpu.async_copy(x_hbm.at[idx_ts], rows_t, sem_t).wait()
    pltpu.sync_copy(rows_t, acc.at[idx_td], add=True)

    def step(ci, k, do_load):
        p = k % 2
        q = (2 + k) % 4
        q_w = k % 4
        q_prev = (1 + k) % 4
        wait_scatter(p, q_w)
        if do_load:
            load_idx(q_w, ci + 2)
        wait_idx(q)
        gather(p, q)
        wait_gather(1 - p, q_prev)
        scatter(1 - p, q_prev)

    load_idx(0, 0)
    load_idx(1, 1)
    wait_idx(0)
    gather(0, 0)
    load_idx(2, 2)
    wait_idx(1)
    gather(1, 1)
    load_idx(3, 3)
    wait_gather(0, 0)
    scatter(0, 0)

    @pl.loop(0, (nchunks - 6) // 4)
    def _(gi):
        for k in range(4):
            step(gi * 4 + 2 + k, k, True)

    step(nchunks - 4, 0, True)
    step(nchunks - 3, 1, True)
    step(nchunks - 2, 2, False)
    step(nchunks - 1, 3, False)

    wait_gather(1, 1)
    scatter(1, 1)
    wait_scatter(0, 0)
    wait_scatter(1, 1)

    plsc.subcore_barrier()
    row0 = s * nps
    pltpu.sync_copy(acc.at[pl.ds(row0, nps)], out_hbm.at[c, pl.ds(row0, nps)])


def _aggregate(x, src, dst):
    n, d = x.shape
    e = src.shape[0]
    epw = e // _NW
    assert epw % _ACHUNK == _ATAIL and (epw // _ACHUNK) % 2 == 0
    z = jnp.zeros((_NPAD // _NS, d), jnp.float32)
    mesh = plsc.VectorSubcoreMesh(core_axis_name="c", subcore_axis_name="s")
    f = pl.kernel(
        functools.partial(_agg_body, epw),
        out_type=jax.ShapeDtypeStruct((_NC, _NPAD, d), jnp.float32),
        mesh=mesh,
        scratch_types=(
            [pltpu.VMEM((_ACHUNK,), jnp.int32)] * 8
            + [pltpu.VMEM((_ACHUNK, d), jnp.float32)] * 2
            + [pltpu.VMEM((_ATAIL,), jnp.int32)] * 2
            + [pltpu.VMEM((_ATAIL, d), jnp.float32)]
            + [pltpu.SemaphoreType.DMA] * 9
            + [pltpu.VMEM_SHARED((_NPAD, d), jnp.float32)]
        ),
    )
    return f(x, src, dst, z)


def _tc_a_body(h_ref, degs_ref, degd_ref, x1_ref, norms_ref):
    n = h_ref.shape[0]
    deg_out = jnp.sum(degs_ref[...], axis=0)[:, None]
    deg_in = jnp.sum(degd_ref[...], axis=0)[:, None]
    norm_src = jnp.where(deg_out > 0, deg_out, 1.0) ** -0.5
    norm_dst = jnp.where(deg_in > 0, deg_in, 1.0) ** -0.5
    x1_ref[...] = h_ref[...] * norm_src
    norms_ref[...] = jnp.concatenate(
        [norm_src, norm_dst, jnp.zeros((n, 6), jnp.float32)], axis=1)


def _tc_a(h, degs, degd):
    return pl.pallas_call(
        _tc_a_body,
        out_shape=[
            jax.ShapeDtypeStruct(h.shape, jnp.float32),
            jax.ShapeDtypeStruct((h.shape[0], 8), jnp.float32),
        ],
    )(h, degs, degd)


def _mm(a, b):
    return lax.dot(a, b, preferred_element_type=jnp.float32)


def _mm_t(a, b):
    return lax.dot_general(a, b, (((0,), (0,)), ((), ())),
                           preferred_element_type=jnp.float32)


def _graph_norm_relu(t, onehot, inv_cnt, gamma, beta, alpha):
    seg = _mm_t(onehot, t)
    mean = seg * inv_cnt
    sub = t - alpha * _mm(onehot, mean)
    var = _mm_t(onehot, sub * sub) * inv_cnt
    std = jnp.sqrt(_mm(onehot, var) + 1e-5)
    return jnp.maximum(gamma * sub / std + beta, 0.0)


def _onehot_and_cnt(gid, g):
    iota = lax.broadcasted_iota(jnp.int32, (1, g), 1)
    onehot = (gid == iota).astype(jnp.float32)
    cnt = jnp.sum(onehot, axis=0)
    inv_cnt = (1.0 / jnp.maximum(cnt, 1.0))[:, None]
    return onehot, inv_cnt


def _tc_b_body(g, aggp_ref, norms_ref, gid_ref, w_ref, b_ref, gamma_ref,
               beta_ref, alpha_ref, h1_ref, x2_ref):
    n = h1_ref.shape[0]
    norm_src = norms_ref[:, 0:1]
    norm_dst = norms_ref[:, 1:2]
    agg = aggp_ref[0, :n] + aggp_ref[1, :n]
    t = _mm(agg * norm_dst, w_ref[...]) + b_ref[...]
    onehot, inv_cnt = _onehot_and_cnt(gid_ref[...], g)
    h1 = _graph_norm_relu(t, onehot, inv_cnt, gamma_ref[...], beta_ref[...],
                          alpha_ref[...])
    h1_ref[...] = h1
    x2_ref[...] = h1 * norm_src


def _tc_b(aggp, norms, gid, w, b, gamma, beta, alpha, g):
    n, d = gid.shape[0], aggp.shape[2]
    return pl.pallas_call(
        functools.partial(_tc_b_body, g),
        out_shape=[
            jax.ShapeDtypeStruct((n, d), jnp.float32),
            jax.ShapeDtypeStruct((n, d), jnp.float32),
        ],
        compiler_params=pltpu.CompilerParams(
            vmem_limit_bytes=100 * 1024 * 1024),
    )(aggp, norms, gid, w, b[None, :], gamma[None, :], beta[None, :],
      alpha[None, :])


def _tc_c_body(g, aggp_ref, norms_ref, gid_ref, h1_ref, w_ref, b_ref,
               gamma_ref, beta_ref, alpha_ref, wp_ref, bp_ref, out_ref):
    n = h1_ref.shape[0]
    norm_dst = norms_ref[:, 1:2]
    agg = aggp_ref[0, :n] + aggp_ref[1, :n]
    t = _mm(agg * norm_dst, w_ref[...]) + b_ref[...]
    onehot, inv_cnt = _onehot_and_cnt(gid_ref[...], g)
    h2 = _graph_norm_relu(t, onehot, inv_cnt, gamma_ref[...], beta_ref[...],
                          alpha_ref[...]) + h1_ref[...]
    pooled = _mm_t(onehot, h2)
    out_ref[...] = _mm(pooled, wp_ref[...]) + bp_ref[...]


def _tc_c(aggp, norms, gid, h1, w, b, gamma, beta, alpha, wp, bp, g):
    d = aggp.shape[2]
    return pl.pallas_call(
        functools.partial(_tc_c_body, g),
        out_shape=jax.ShapeDtypeStruct((g, d), jnp.float32),
        compiler_params=pltpu.CompilerParams(
            vmem_limit_bytes=100 * 1024 * 1024),
    )(aggp, norms, gid, h1, w, b[None, :], gamma[None, :], beta[None, :],
      alpha[None, :], wp, bp[None, :])


def kernel(h, edge_index, graph_ids, W1, b1, gamma1, beta1, alpha1,
           W2, b2, gamma2, beta2, alpha2, Wp, bp):
    n, d = h.shape
    g = 64
    src = edge_index[0]
    dst = edge_index[1]
    gid = graph_ids[:, None]

    degs, degd = _degrees(src, dst, n)
    x1, norms = _tc_a(h, degs, degd)
    aggp1 = _aggregate(x1, src, dst)
    h1, x2 = _tc_b(aggp1, norms, gid, W1, b1, gamma1, beta1, alpha1, g)
    aggp2 = _aggregate(x2, src, dst)
    return _tc_c(aggp2, norms, gid, h1, W2, b2, gamma2, beta2, alpha2, Wp,
                 bp, g)


# speedup vs baseline: 1.0019x; 1.0019x over previous
import functools
import jax
import jax.numpy as jnp
from jax import lax
from jax.experimental import pallas as pl
from jax.experimental.pallas import tpu as pltpu
from jax.experimental.pallas import tpu_sc as plsc

_NC = 2
_NS = 16
_NW = _NC * _NS
_NPAD = 10240


_DCHUNK = 400


def _deg_body(n, epw, src_hbm, dst_hbm, z_hbm, outs_hbm, outd_hbm,
              ia0, ia1, ib0, ib1, sa0, sa1, sb0, sb1,
              hist_s, hist_d):
    c = lax.axis_index("c")
    s = lax.axis_index("s")
    wid = s * _NC + c

    pltpu.sync_copy(z_hbm, hist_s)
    pltpu.sync_copy(z_hbm, hist_d)

    base = wid * epw
    nchunks = epw // _DCHUNK
    ia = (ia0, ia1)
    ib = (ib0, ib1)
    sa = (sa0, sa1)
    sb = (sb0, sb1)
    ones = jnp.full((16,), 1.0, jnp.float32)

    def load(p, ci):
        off = pl.multiple_of(base + ci * _DCHUNK, 8)
        pltpu.async_copy(src_hbm.at[pl.ds(off, _DCHUNK)], ia[p], sa[p])
        pltpu.async_copy(dst_hbm.at[pl.ds(off, _DCHUNK)], ib[p], sb[p])

    def wait_load(p):
        pltpu.make_async_copy(src_hbm.at[pl.ds(0, _DCHUNK)], ia[p], sa[p]).wait()
        pltpu.make_async_copy(dst_hbm.at[pl.ds(0, _DCHUNK)], ib[p], sb[p]).wait()

    def process(p):
        for j in range(_DCHUNK // 16):
            plsc.addupdate_scatter(hist_s, [ia[p][pl.ds(j * 16, 16)]], ones)
            plsc.addupdate_scatter(hist_d, [ib[p][pl.ds(j * 16, 16)]], ones)

    load(0, 0)

    @pl.loop(0, (nchunks - 1) // 2)
    def _(gi):
        for k in range(2):
            ci = gi * 2 + k
            wait_load(k)
            load(1 - k, ci + 1)
            process(k)

    wait_load(0)
    process(0)

    pltpu.sync_copy(hist_s, outs_hbm.at[pl.ds(wid * n, n)])
    pltpu.sync_copy(hist_d, outd_hbm.at[pl.ds(wid * n, n)])


def _degrees(src, dst, n):
    e = src.shape[0]
    epw = e // _NW
    assert epw % _DCHUNK == 0 and (epw // _DCHUNK) % 2 == 1
    z = jnp.zeros((n,), jnp.float32)
    mesh = plsc.VectorSubcoreMesh(core_axis_name="c", subcore_axis_name="s")
    f = pl.kernel(
        functools.partial(_deg_body, n, epw),
        out_type=[
            jax.ShapeDtypeStruct((_NW * n,), jnp.float32),
            jax.ShapeDtypeStruct((_NW * n,), jnp.float32),
        ],
        mesh=mesh,
        scratch_types=[
            pltpu.VMEM((_DCHUNK,), jnp.int32),
            pltpu.VMEM((_DCHUNK,), jnp.int32),
            pltpu.VMEM((_DCHUNK,), jnp.int32),
            pltpu.VMEM((_DCHUNK,), jnp.int32),
            pltpu.SemaphoreType.DMA,
            pltpu.SemaphoreType.DMA,
            pltpu.SemaphoreType.DMA,
            pltpu.SemaphoreType.DMA,
            pltpu.VMEM((n,), jnp.float32),
            pltpu.VMEM((n,), jnp.float32),
        ],
        compiler_params=pltpu.CompilerParams(needs_layout_passes=False),
    )
    outs, outd = f(src, dst, z)
    return outs.reshape(_NW, n), outd.reshape(_NW, n)


_ACHUNK = 128
_ATAIL = 16


def _agg_body(epw, x_hbm, src_hbm, dst_hbm, z_hbm, out_hbm,
              idx_s0, idx_s1, idx_s2, idx_s3, idx_d0, idx_d1, idx_d2, idx_d3,
              rows0, rows1, idx_ts, idx_td, rows_t,
              sem_g0, sem_g1, sem_s0, sem_s1,
              sem_i0, sem_i1, sem_i2, sem_i3, sem_t, acc):
    c = lax.axis_index("c")
    s = lax.axis_index("s")
    wid = s * _NC + c
    nps = _NPAD // _NS

    pltpu.sync_copy(z_hbm, acc.at[pl.ds(s * nps, nps)])
    plsc.subcore_barrier()

    base = wid * epw
    nchunks = epw // _ACHUNK
    idx_s = (idx_s0, idx_s1, idx_s2, idx_s3)
    idx_d = (idx_d0, idx_d1, idx_d2, idx_d3)
    rows = (rows0, rows1)
    sem_g = (sem_g0, sem_g1)
    sem_s = (sem_s0, sem_s1)
    sem_i = (sem_i0, sem_i1, sem_i2, sem_i3)

    def load_idx(q, ci):
        off = pl.multiple_of(base + ci * _ACHUNK, 8)
        pltpu.async_copy(src_hbm.at[pl.ds(off, _ACHUNK)], idx_s[q], sem_i[q])
        pltpu.async_copy(dst_hbm.at[pl.ds(off, _ACHUNK)], idx_d[q], sem_i[q])

    def wait_idx(q):
        pltpu.make_async_copy(src_hbm.at[pl.ds(0, _ACHUNK)], idx_s[q],
                              sem_i[q]).wait()
        pltpu.make_async_copy(dst_hbm.at[pl.ds(0, _ACHUNK)], idx_d[q],
                              sem_i[q]).wait()

    def gather(p, q):
        pltpu.async_copy(x_hbm.at[idx_s[q]], rows[p], sem_g[p])

    def wait_gather(p, q):
        pltpu.make_async_copy(x_hbm.at[idx_s[q]], rows[p], sem_g[p]).wait()

    def scatter(p, q):
        pltpu.async_copy(rows[p], acc.at[idx_d[q]], sem_s[p], add=True)

    def wait_scatter(p, q):
        pltpu.make_async_copy(rows[p], acc.at[idx_d[q]], sem_s[p]).wait()

    toff = pl.multiple_of(base + nchunks * _ACHUNK, 8)
    pltpu.sync_copy(src_hbm.at[pl.ds(toff, _ATAIL)], idx_ts)
    pltpu.sync_copy(dst_hbm.at[pl.ds(toff, _ATAIL)], idx_td)
    pltpu.async_copy(x_hbm.at[idx_ts], rows_t, sem_t).wait()
    pltpu.sync_copy(rows_t, acc.at[idx_td], add=True)

    def step(ci, k, do_load):
        p = k % 2
        q = (2 + k) % 4
        q_w = k % 4
        q_prev = (1 + k) % 4
        wait_scatter(p, q_w)
        if do_load:
            load_idx(q_w, ci + 2)
        wait_idx(q)
        gather(p, q)
        wait_gather(1 - p, q_prev)
        scatter(1 - p, q_prev)

    load_idx(0, 0)
    load_idx(1, 1)
    wait_idx(0)
    gather(0, 0)
    load_idx(2, 2)
    wait_idx(1)
    gather(1, 1)
    load_idx(3, 3)
    wait_gather(0, 0)
    scatter(0, 0)

    @pl.loop(0, (nchunks - 6) // 4)
    def _(gi):
        for k in range(4):
            step(gi * 4 + 2 + k, k, True)

    step(nchunks - 4, 0, True)
    step(nchunks - 3, 1, True)
    step(nchunks - 2, 2, False)
    step(nchunks - 1, 3, False)

    wait_gather(1, 1)
    scatter(1, 1)
    wait_scatter(0, 0)
    wait_scatter(1, 1)

    plsc.subcore_barrier()
    row0 = s * nps
    pltpu.sync_copy(acc.at[pl.ds(row0, nps)], out_hbm.at[c, pl.ds(row0, nps)])


def _aggregate(x, src, dst):
    n, d = x.shape
    e = src.shape[0]
    epw = e // _NW
    assert epw % _ACHUNK == _ATAIL and (epw // _ACHUNK) % 2 == 0
    z = jnp.zeros((_NPAD // _NS, d), jnp.float32)
    mesh = plsc.VectorSubcoreMesh(core_axis_name="c", subcore_axis_name="s")
    f = pl.kernel(
        functools.partial(_agg_body, epw),
        out_type=jax.ShapeDtypeStruct((_NC, _NPAD, d), jnp.float32),
        mesh=mesh,
        scratch_types=(
            [pltpu.VMEM((_ACHUNK,), jnp.int32)] * 8
            + [pltpu.VMEM((_ACHUNK, d), jnp.float32)] * 2
            + [pltpu.VMEM((_ATAIL,), jnp.int32)] * 2
            + [pltpu.VMEM((_ATAIL, d), jnp.float32)]
            + [pltpu.SemaphoreType.DMA] * 9
            + [pltpu.VMEM_SHARED((_NPAD, d), jnp.float32)]
        ),
    )
    return f(x, src, dst, z)


def _tc_a_body(h_ref, degs_ref, degd_ref, x1_ref, norms_ref):
    n = h_ref.shape[0]
    deg_out = jnp.sum(degs_ref[...], axis=0)[:, None]
    deg_in = jnp.sum(degd_ref[...], axis=0)[:, None]
    norm_src = jnp.where(deg_out > 0, deg_out, 1.0) ** -0.5
    norm_dst = jnp.where(deg_in > 0, deg_in, 1.0) ** -0.5
    x1_ref[...] = h_ref[...] * norm_src
    norms_ref[...] = jnp.concatenate(
        [norm_src, norm_dst, jnp.zeros((n, 6), jnp.float32)], axis=1)


def _tc_a(h, degs, degd):
    return pl.pallas_call(
        _tc_a_body,
        out_shape=[
            jax.ShapeDtypeStruct(h.shape, jnp.float32),
            jax.ShapeDtypeStruct((h.shape[0], 8), jnp.float32),
        ],
    )(h, degs, degd)


def _mm(a, b):
    return lax.dot(a, b, preferred_element_type=jnp.float32)


def _mm_t(a, b):
    return lax.dot_general(a, b, (((0,), (0,)), ((), ())),
                           preferred_element_type=jnp.float32)


def _graph_norm_relu(t, onehot, inv_cnt, gamma, beta, alpha):
    seg = _mm_t(onehot, t)
    mean = seg * inv_cnt
    sub = t - alpha * _mm(onehot, mean)
    var = _mm_t(onehot, sub * sub) * inv_cnt
    std = jnp.sqrt(_mm(onehot, var) + 1e-5)
    return jnp.maximum(gamma * sub / std + beta, 0.0)


def _onehot_and_cnt(gid, g):
    iota = lax.broadcasted_iota(jnp.int32, (1, g), 1)
    onehot = (gid == iota).astype(jnp.float32)
    cnt = jnp.sum(onehot, axis=0)
    inv_cnt = (1.0 / jnp.maximum(cnt, 1.0))[:, None]
    return onehot, inv_cnt


def _tc_b_body(g, aggp_ref, norms_ref, gid_ref, w_ref, b_ref, gamma_ref,
               beta_ref, alpha_ref, h1_ref, x2_ref):
    n = h1_ref.shape[0]
    norm_src = norms_ref[:, 0:1]
    norm_dst = norms_ref[:, 1:2]
    agg = aggp_ref[0, :n] + aggp_ref[1, :n]
    t = _mm(agg * norm_dst, w_ref[...]) + b_ref[...]
    onehot, inv_cnt = _onehot_and_cnt(gid_ref[...], g)
    h1 = _graph_norm_relu(t, onehot, inv_cnt, gamma_ref[...], beta_ref[...],
                          alpha_ref[...])
    h1_ref[...] = h1
    x2_ref[...] = h1 * norm_src


def _tc_b(aggp, norms, gid, w, b, gamma, beta, alpha, g):
    n, d = gid.shape[0], aggp.shape[2]
    return pl.pallas_call(
        functools.partial(_tc_b_body, g),
        out_shape=[
            jax.ShapeDtypeStruct((n, d), jnp.float32),
            jax.ShapeDtypeStruct((n, d), jnp.float32),
        ],
        compiler_params=pltpu.CompilerParams(
            vmem_limit_bytes=100 * 1024 * 1024),
    )(aggp, norms, gid, w, b[None, :], gamma[None, :], beta[None, :],
      alpha[None, :])


def _tc_c_body(g, aggp_ref, norms_ref, gid_ref, h1_ref, w_ref, b_ref,
               gamma_ref, beta_ref, alpha_ref, wp_ref, bp_ref, out_ref):
    n = h1_ref.shape[0]
    norm_dst = norms_ref[:, 1:2]
    agg = aggp_ref[0, :n] + aggp_ref[1, :n]
    t = _mm(agg * norm_dst, w_ref[...]) + b_ref[...]
    onehot, inv_cnt = _onehot_and_cnt(gid_ref[...], g)
    h2 = _graph_norm_relu(t, onehot, inv_cnt, gamma_ref[...], beta_ref[...],
                          alpha_ref[...]) + h1_ref[...]
    pooled = _mm_t(onehot, h2)
    out_ref[...] = _mm(pooled, wp_ref[...]) + bp_ref[...]


def _tc_c(aggp, norms, gid, h1, w, b, gamma, beta, alpha, wp, bp, g):
    d = aggp.shape[2]
    return pl.pallas_call(
        functools.partial(_tc_c_body, g),
        out_shape=jax.ShapeDtypeStruct((g, d), jnp.float32),
        compiler_params=pltpu.CompilerParams(
            vmem_limit_bytes=100 * 1024 * 1024),
    )(aggp, norms, gid, h1, w, b[None, :], gamma[None, :], beta[None, :],
      alpha[None, :], wp, bp[None, :])


def kernel(h, edge_index, graph_ids, W1, b1, gamma1, beta1, alpha1,
           W2, b2, gamma2, beta2, alpha2, Wp, bp):
    n, d = h.shape
    g = 64
    src = edge_index[0]
    dst = edge_index[1]
    gid = graph_ids[:, None]

    degs, degd = _degrees(src, dst, n)
    x1, norms = _tc_a(h, degs, degd)
    aggp1 = _aggregate(x1, src, dst)
    h1, x2 = _tc_b(aggp1, norms, gid, W1, b1, gamma1, beta1, alpha1, g)
    aggp2 = _aggregate(x2, src, dst)
    return _tc_c(aggp2, norms, gid, h1, W2, b2, gamma2, beta2, alpha2, Wp,
                 bp, g)
